# Initial kernel scaffold; baseline (speedup 1.0000x reference)
import jax
import jax.numpy as jnp
from jax.experimental import pallas as pl


def kernel(x, tables):
    raise NotImplementedError("write your pallas kernel here")


# speedup vs baseline: 1.3000x; 1.3000x over previous
import functools

import jax
import jax.numpy as jnp
from jax import lax
from jax.experimental import pallas as pl
from jax.experimental.pallas import tpu as pltpu
from jax.experimental.pallas import tpu_sc as plsc

B, T, S, F = 1024, 20, 2, 26
V, D = 100000, 32
N = B * T * S
M = N * F
NW = 32
PER_W = M // NW
GROUP = 128
NG = 13
BATCH = GROUP * NG
NB = PER_W // BATCH
L = 16


def _body(xf, tab, out, idxb, offb, rows, gsem):
    wid = lax.axis_index("s") * 2 + lax.axis_index("c")
    wbase = wid * PER_W

    @pl.loop(0, BATCH // L)
    def _off(k):
        pos = k * L + lax.iota(jnp.int32, L)
        offb[pl.ds(k * L, L)] = (pos % F) * V

    @pl.loop(0, NB)
    def _batch(b):
        base = wbase + b * BATCH
        pltpu.sync_copy(xf.at[pl.ds(base, BATCH)], idxb)

        @pl.loop(0, BATCH // L)
        def _add(k):
            idxb[pl.ds(k * L, L)] = idxb[pl.ds(k * L, L)] + offb[pl.ds(k * L, L)]

        copies = [
            pltpu.async_copy(
                tab.at[idxb.at[pl.ds(g * GROUP, GROUP)]],
                rows.at[pl.ds(g * GROUP, GROUP)],
                gsem,
            )
            for g in range(NG)
        ]
        for c in copies:
            c.wait()
        pltpu.sync_copy(rows, out.at[pl.ds(base, BATCH)])


@jax.jit
def _embed(xf, tab):
    mesh = plsc.VectorSubcoreMesh(core_axis_name="c", subcore_axis_name="s")
    return pl.kernel(
        _body,
        out_type=jax.ShapeDtypeStruct((M, D), jnp.float32),
        mesh=mesh,
        scratch_types=[
            pltpu.VMEM((BATCH,), jnp.int32),
            pltpu.VMEM((BATCH,), jnp.int32),
            pltpu.VMEM((BATCH, D), jnp.float32),
            pltpu.SemaphoreType.DMA,
        ],
        compiler_params=pltpu.CompilerParams(use_tc_tiling_on_sc=False),
    )(xf, tab)


def kernel(x, tables):
    xf = x.reshape(M)
    tab = tables.reshape(F * V, D)
    out = _embed(xf, tab)
    return out.reshape(B, T, S, F * D)


# speedup vs baseline: 1.3180x; 1.0138x over previous
import functools

import jax
import jax.numpy as jnp
from jax import lax
from jax.experimental import pallas as pl
from jax.experimental.pallas import tpu as pltpu
from jax.experimental.pallas import tpu_sc as plsc

B, T, S, F = 1024, 20, 2, 26
V, D = 100000, 32
N = B * T * S
M = N * F
NW = 32
PER_W = M // NW
GROUP = 128
NG = 13
BATCH = GROUP * NG
NB = PER_W // BATCH
L = 16


def _body(xf, tab, out, idx_a, idx_b, offb, rows_a, rows_b,
          gsem_a, gsem_b, osem_a, osem_b):
    wid = lax.axis_index("s") * 2 + lax.axis_index("c")
    wbase = wid * PER_W

    @pl.loop(0, BATCH // L)
    def _off(k):
        pos = k * L + lax.iota(jnp.int32, L)
        offb[pl.ds(k * L, L)] = (pos % F) * V

    def load_compute(b, idxb):
        pltpu.sync_copy(xf.at[pl.ds(wbase + b * BATCH, BATCH)], idxb)

        @pl.loop(0, BATCH // L)
        def _add(k):
            idxb[pl.ds(k * L, L)] = idxb[pl.ds(k * L, L)] + offb[pl.ds(k * L, L)]

    def fire_gathers(idxb, rows, gsem):
        for g in range(NG):
            pltpu.async_copy(
                tab.at[idxb.at[pl.ds(g * GROUP, GROUP)]],
                rows.at[pl.ds(g * GROUP, GROUP)],
                gsem,
            )

    def wait_gathers(rows, gsem):
        pltpu.make_async_copy(out.at[pl.ds(0, BATCH)], rows, gsem).wait()

    def fire_out(b, rows, osem):
        pltpu.async_copy(rows, out.at[pl.ds(wbase + b * BATCH, BATCH)], osem)

    def wait_out(rows, osem):
        pltpu.make_async_copy(rows, out.at[pl.ds(0, BATCH)], osem).wait()

    load_compute(0, idx_a)
    fire_gathers(idx_a, rows_a, gsem_a)
    load_compute(1, idx_b)
    fire_gathers(idx_b, rows_b, gsem_b)

    @pl.loop(0, (NB - 2) // 2)
    def _iter(i):
        b = 2 * i
        wait_gathers(rows_a, gsem_a)
        fire_out(b, rows_a, osem_a)
        load_compute(b + 2, idx_a)
        wait_out(rows_a, osem_a)
        fire_gathers(idx_a, rows_a, gsem_a)

        wait_gathers(rows_b, gsem_b)
        fire_out(b + 1, rows_b, osem_b)
        load_compute(b + 3, idx_b)
        wait_out(rows_b, osem_b)
        fire_gathers(idx_b, rows_b, gsem_b)

    wait_gathers(rows_a, gsem_a)
    fire_out(NB - 2, rows_a, osem_a)
    wait_gathers(rows_b, gsem_b)
    fire_out(NB - 1, rows_b, osem_b)
    wait_out(rows_a, osem_a)
    wait_out(rows_b, osem_b)


@jax.jit
def _embed(xf, tab):
    mesh = plsc.VectorSubcoreMesh(core_axis_name="c", subcore_axis_name="s")
    return pl.kernel(
        _body,
        out_type=jax.ShapeDtypeStruct((M, D), jnp.float32),
        mesh=mesh,
        scratch_types=[
            pltpu.VMEM((BATCH,), jnp.int32),
            pltpu.VMEM((BATCH,), jnp.int32),
            pltpu.VMEM((BATCH,), jnp.int32),
            pltpu.VMEM((BATCH, D), jnp.float32),
            pltpu.VMEM((BATCH, D), jnp.float32),
            pltpu.SemaphoreType.DMA,
            pltpu.SemaphoreType.DMA,
            pltpu.SemaphoreType.DMA,
            pltpu.SemaphoreType.DMA,
        ],
        compiler_params=pltpu.CompilerParams(use_tc_tiling_on_sc=False),
    )(xf, tab)


def kernel(x, tables):
    xf = x.reshape(M)
    tab = tables.reshape(F * V, D)
    out = _embed(xf, tab)
    return out.reshape(B, T, S, F * D)
